# baseline (speedup 1.0000x reference)
import functools

import jax
import jax.numpy as jnp
from jax import lax
from jax.experimental import pallas as pl
from jax.experimental.pallas import tpu as pltpu
from jax.experimental.pallas import tpu_sc as plsc

N = 10000
E = 320000
FEATS = 128
HIDDEN = 64
C = 64
DEPTH = 10

NSC = 2
NT = 16
NPAD = 10240
ROWS_SC = NPAD // NSC
TROWS = ROWS_SC // NT
NACC = ROWS_SC + 128
ZR = NACC // NT

EPT = E // NT
STAGE = 2000
CHUNK = 128
CHMAX = (EPT + CHUNK - 1) // CHUNK + 1
EBUF = CHMAX * CHUNK

NTR = 1024
TPT = NTR // NT
TRPAD = 4 * NPAD


def _i16():
    return jnp.arange(16, dtype=jnp.int32)


def _mlp_body(x_ref, w1_ref, b1_ref, w2_ref, b2_ref, p_ref):
    h = jnp.maximum(x_ref[...] @ w1_ref[...] + b1_ref[...][None, :], 0.0)
    logits = h @ w2_ref[...] + b2_ref[...][None, :]
    m = jnp.max(logits, axis=1, keepdims=True)
    e = jnp.exp(logits - m)
    p_ref[...] = e / jnp.sum(e, axis=1, keepdims=True)


def _mlp_softmax(x, W1, b1, W2, b2):
    blk = 2000
    return pl.pallas_call(
        _mlp_body,
        grid=(N // blk,),
        in_specs=[
            pl.BlockSpec((blk, FEATS), lambda i: (i, 0)),
            pl.BlockSpec((FEATS, HIDDEN), lambda i: (0, 0)),
            pl.BlockSpec((HIDDEN,), lambda i: (0,)),
            pl.BlockSpec((HIDDEN, C), lambda i: (0, 0)),
            pl.BlockSpec((C,), lambda i: (0,)),
        ],
        out_specs=pl.BlockSpec((blk, C), lambda i: (i, 0)),
        out_shape=jax.ShapeDtypeStruct((N, C), jnp.float32),
    )(x, W1, b1, W2, b2)


def _trans_body(p_ref, t_ref, h0b_ref, b2_ref):
    h0b = p_ref[...] + t_ref[...]
    h0b_ref[...] = h0b
    b2_ref[...] = 0.1 * h0b


def _transition(p_pad, t10):
    blk = 2560
    return pl.pallas_call(
        _trans_body,
        grid=(NPAD // blk,),
        in_specs=[
            pl.BlockSpec((blk, C), lambda i: (i, 0)),
            pl.BlockSpec((blk, C), lambda i: (i, 0)),
        ],
        out_specs=[
            pl.BlockSpec((blk, C), lambda i: (i, 0)),
            pl.BlockSpec((blk, C), lambda i: (i, 0)),
        ],
        out_shape=[
            jax.ShapeDtypeStruct((NPAD, C), jnp.float32),
            jax.ShapeDtypeStruct((NPAD, C), jnp.float32),
        ],
    )(p_pad, t10)


def _log1p_body(t_ref, o_ref):
    o_ref[...] = jnp.log(t_ref[...] + 1.0)


def _finalize(t):
    blk = 2000
    return pl.pallas_call(
        _log1p_body,
        grid=(N // blk,),
        in_specs=[pl.BlockSpec((blk, C), lambda i: (i, 0))],
        out_specs=pl.BlockSpec((blk, C), lambda i: (i, 0)),
        out_shape=jax.ShapeDtypeStruct((N, C), jnp.float32),
    )(t)


_MESH = plsc.VectorSubcoreMesh(core_axis_name="c", subcore_axis_name="s")


def _sc_setup_body(
    src_hbm, dst_hbm, tr_hbm, lab_hbm, p_hbm,
    esrc_hbm, edst_hbm, nch_hbm, scale1_hbm, scale2_hbm, bias1_hbm, t0_hbm,
    stage_s, stage_d, out_src, out_dst, ones_e, zsmall, pblk, bblk,
    trows, tidx, tloc, tlab, ones64, deg_t, msk_t, s1_t, s2_t, n16,
    degsp, masksp, bias1sp, sem,
):
    c = lax.axis_index("c")
    s = lax.axis_index("s")
    w = c * NT + s
    lo = s * TROWS
    g0 = c * ROWS_SC + s * TROWS
    sc_lo = c * ROWS_SC

    z16 = jnp.zeros((16,), jnp.float32)
    o16 = jnp.ones((16,), jnp.float32)
    i16 = _i16()

    def fill_z(j, _):
        zsmall[pl.ds(j * 16, 16)] = z16
        return 0
    lax.fori_loop(0, 336 // 16, fill_z, 0)

    def fill_bblk(r, _):
        for g in range(4):
            bblk[r, pl.ds(g * 16, 16)] = z16
        return 0
    lax.fori_loop(0, 16, fill_bblk, 0)

    def fill_ones(j, _):
        ones_e[pl.ds(j * 16, 16)] = o16
        return 0
    lax.fori_loop(0, EBUF // 16, fill_ones, 0)

    for g in range(4):
        ones64[pl.ds(g * 16, 16)] = o16

    def fill_pad(j, _):
        lane = j * 16 + i16
        out_src[pl.ds(j * 16, 16)] = lane % N
        out_dst[pl.ds(j * 16, 16)] = ROWS_SC + (lane % 128)
        return 0
    lax.fori_loop(0, EBUF // 16, fill_pad, 0)

    pltpu.sync_copy(zsmall.at[pl.ds(0, ZR)], degsp.at[pl.ds(s * ZR, ZR)])
    pltpu.sync_copy(zsmall.at[pl.ds(0, ZR)], masksp.at[pl.ds(s * ZR, ZR)])

    def zero_b1(j, _):
        pltpu.sync_copy(bblk, bias1sp.at[pl.ds(s * ZR + j * 16, 16)])
        return 0
    lax.fori_loop(0, ZR // 16, zero_b1, 0)
    pltpu.sync_copy(bblk.at[pl.ds(0, ZR - 20 * 16)],
                    bias1sp.at[pl.ds(s * ZR + 20 * 16, ZR - 20 * 16)])
    plsc.subcore_barrier()

    def pass_body(k, cnt):
        pltpu.sync_copy(src_hbm.at[pl.ds(s * EPT + k * STAGE, STAGE)], stage_s)
        pltpu.sync_copy(dst_hbm.at[pl.ds(s * EPT + k * STAGE, STAGE)], stage_d)

        def grp(gi, cnt):
            sv = stage_s[pl.ds(gi * 16, 16)]
            dv = stage_d[pl.ds(gi * 16, 16)]
            m = (dv >= sc_lo) & (dv < sc_lo + ROWS_SC)
            mi = m.astype(jnp.int32)
            pos = cnt + plsc.cumsum(mi) - mi
            plsc.store_scatter(out_src, [pos], sv, mask=m)
            plsc.store_scatter(out_dst, [pos], dv - sc_lo, mask=m)
            npop = plsc.all_reduce_population_count(m)
            return cnt + npop[0]
        return lax.fori_loop(0, STAGE // 16, grp, cnt)

    cnt = lax.fori_loop(0, EPT // STAGE, pass_body, jnp.int32(0))
    nch = (cnt + CHUNK - 1) // CHUNK

    pltpu.sync_copy(ones_e, degsp.at[out_dst], add=True)

    pltpu.sync_copy(tr_hbm.at[pl.ds(s * TPT, TPT)], tidx)
    pltpu.sync_copy(lab_hbm.at[pl.ds(s * TPT, TPT)], tlab)
    for g in range(4):
        tv = tidx[pl.ds(g * 16, 16)]
        lv = tv - sc_lo
        valid = (lv >= 0) & (lv < ROWS_SC)
        spread = g * 16 + i16
        tloc[pl.ds(g * 16, 16)] = jnp.where(valid, lv, ROWS_SC + (spread % 128))
        tidx[pl.ds(g * 16, 16)] = jnp.where(tv < NPAD, tv, spread)

    pltpu.async_copy(p_hbm.at[tidx], trows, sem).wait()

    def neg_row(j, _):
        for g in range(4):
            trows[j, pl.ds(g * 16, 16)] = -trows[j, pl.ds(g * 16, 16)]
        return 0
    lax.fori_loop(0, TPT, neg_row, 0)

    for g in range(4):
        jv = g * 16 + i16
        lv16 = tlab[pl.ds(g * 16, 16)]
        plsc.addupdate_scatter(trows, [jv, lv16], o16)

    pltpu.sync_copy(ones64, masksp.at[tloc], add=True)
    pltpu.sync_copy(trows, bias1sp.at[tloc], add=True)
    plsc.subcore_barrier()

    pltpu.sync_copy(degsp.at[pl.ds(lo, TROWS)], deg_t)
    pltpu.sync_copy(masksp.at[pl.ds(lo, TROWS)], msk_t)

    def scales(j, _):
        d = deg_t[pl.ds(j * 16, 16)]
        mk = msk_t[pl.ds(j * 16, 16)]
        dinv = 1.0 / jnp.maximum(d, 1.0)
        s1_t[pl.ds(j * 16, 16)] = dinv * (1.0 - mk)
        s2_t[pl.ds(j * 16, 16)] = 0.9 * dinv
        return 0
    lax.fori_loop(0, TROWS // 16, scales, 0)

    pltpu.sync_copy(s1_t, scale1_hbm.at[pl.ds(g0, TROWS)])
    pltpu.sync_copy(s2_t, scale2_hbm.at[pl.ds(g0, TROWS)])

    pltpu.sync_copy(bias1sp.at[pl.ds(lo, TROWS)], bias1_hbm.at[pl.ds(g0, TROWS)])

    def t0_blk(rb, _):
        pltpu.sync_copy(p_hbm.at[pl.ds(g0 + rb * 16, 16)], pblk)
        pltpu.sync_copy(bias1sp.at[pl.ds(lo + rb * 16, 16)], bblk)
        m16 = msk_t[pl.ds(rb * 16, 16)]
        for j in range(16):
            sc0 = 1.0 - m16[j]
            for g in range(4):
                pblk[j, pl.ds(g * 16, 16)] = (
                    bblk[j, pl.ds(g * 16, 16)]
                    - pblk[j, pl.ds(g * 16, 16)] * sc0
                )
        pltpu.sync_copy(pblk, t0_hbm.at[pl.ds(g0 + rb * 16, 16)])
        return 0
    lax.fori_loop(0, TROWS // 16, t0_blk, 0)

    n16[...] = jnp.full((16,), nch, jnp.int32)
    pltpu.sync_copy(n16, nch_hbm.at[w])
    pltpu.sync_copy(out_src, esrc_hbm.at[w])
    pltpu.sync_copy(out_dst, edst_hbm.at[w])


_sc_setup = pl.kernel(
    _sc_setup_body,
    out_type=[
        jax.ShapeDtypeStruct((NSC * NT, EBUF), jnp.int32),
        jax.ShapeDtypeStruct((NSC * NT, EBUF), jnp.int32),
        jax.ShapeDtypeStruct((NSC * NT, 16), jnp.int32),
        jax.ShapeDtypeStruct((NPAD,), jnp.float32),
        jax.ShapeDtypeStruct((NPAD,), jnp.float32),
        jax.ShapeDtypeStruct((NPAD, C), jnp.float32),
        jax.ShapeDtypeStruct((NPAD, C), jnp.float32),
    ],
    mesh=_MESH,
    compiler_params=pltpu.CompilerParams(needs_layout_passes=False, use_tc_tiling_on_sc=False),
    scratch_types=[
        pltpu.VMEM((STAGE,), jnp.int32),
        pltpu.VMEM((STAGE,), jnp.int32),
        pltpu.VMEM((EBUF,), jnp.int32),
        pltpu.VMEM((EBUF,), jnp.int32),
        pltpu.VMEM((EBUF,), jnp.float32),
        pltpu.VMEM((336,), jnp.float32),
        pltpu.VMEM((16, C), jnp.float32),
        pltpu.VMEM((16, C), jnp.float32),
        pltpu.VMEM((TPT, C), jnp.float32),
        pltpu.VMEM((TPT,), jnp.int32),
        pltpu.VMEM((TPT,), jnp.int32),
        pltpu.VMEM((TPT,), jnp.int32),
        pltpu.VMEM((TPT,), jnp.float32),
        pltpu.VMEM((TROWS,), jnp.float32),
        pltpu.VMEM((TROWS,), jnp.float32),
        pltpu.VMEM((TROWS,), jnp.float32),
        pltpu.VMEM((TROWS,), jnp.float32),
        pltpu.VMEM((16,), jnp.int32),
        pltpu.VMEM_SHARED((NACC,), jnp.float32),
        pltpu.VMEM_SHARED((NACC,), jnp.float32),
        pltpu.VMEM_SHARED((NACC, C), jnp.float32),
        pltpu.SemaphoreType.DMA,
    ],
)


def _sc_conv_body(
    t_hbm, esrc_hbm, edst_hbm, nch_hbm, scale_hbm, bias_hbm,
    tout_hbm,
    esrc_v, edst_v, rows0, rows1, scale_v, n16,
    accsp, tsp, sem0, sem1,
):
    c = lax.axis_index("c")
    s = lax.axis_index("s")
    w = c * NT + s
    lo = s * TROWS
    g0 = c * ROWS_SC + s * TROWS

    z16 = jnp.zeros((16,), jnp.float32)

    def fill_z(r, _):
        for g in range(4):
            rows0[r, pl.ds(g * 16, 16)] = z16
        return 0
    lax.fori_loop(0, CHUNK, fill_z, 0)
    pltpu.sync_copy(rows0, accsp.at[pl.ds(s * ZR, CHUNK)])
    pltpu.sync_copy(rows0, accsp.at[pl.ds(s * ZR + CHUNK, CHUNK)])
    pltpu.sync_copy(rows0.at[pl.ds(0, ZR - 2 * CHUNK)],
                    accsp.at[pl.ds(s * ZR + 2 * CHUNK, ZR - 2 * CHUNK)])

    pltpu.sync_copy(t_hbm.at[pl.ds(s * (NPAD // NT), NPAD // NT)],
                    tsp.at[pl.ds(s * (NPAD // NT), NPAD // NT)])

    pltpu.sync_copy(esrc_hbm.at[w], esrc_v)
    pltpu.sync_copy(edst_hbm.at[w], edst_v)
    pltpu.sync_copy(nch_hbm.at[w], n16)
    pltpu.sync_copy(scale_hbm.at[pl.ds(g0, TROWS)], scale_v)
    nch = lax.reduce_max(n16[...], (0,))
    plsc.subcore_barrier()

    @pl.when(nch > 0)
    def _():
        pltpu.async_copy(tsp.at[esrc_v.at[0]], rows0, sem0)

    @pl.when(nch > 1)
    def _():
        pltpu.async_copy(tsp.at[esrc_v.at[1]], rows1, sem1)

    def pair(p2, _):
        c0 = p2 * 2
        c1 = c0 + 1

        @pl.when(c0 < nch)
        def _():
            pltpu.make_async_copy(tsp.at[esrc_v.at[c0]], rows0, sem0).wait()
            pltpu.sync_copy(rows0, accsp.at[edst_v.at[c0]], add=True)

            @pl.when(c0 + 2 < nch)
            def _():
                pltpu.async_copy(tsp.at[esrc_v.at[c0 + 2]], rows0, sem0)

        @pl.when(c1 < nch)
        def _():
            pltpu.make_async_copy(tsp.at[esrc_v.at[c1]], rows1, sem1).wait()
            pltpu.sync_copy(rows1, accsp.at[edst_v.at[c1]], add=True)

            @pl.when(c1 + 2 < nch)
            def _():
                pltpu.async_copy(tsp.at[esrc_v.at[c1 + 2]], rows1, sem1)

        return 0

    lax.fori_loop(0, (nch + 1) // 2, pair, 0)
    plsc.subcore_barrier()

    for o, szb in ((0, 128), (128, 128), (256, 64)):
        pltpu.sync_copy(accsp.at[pl.ds(lo + o, szb)], rows0.at[pl.ds(0, szb)])
        pltpu.sync_copy(bias_hbm.at[pl.ds(g0 + o, szb)], rows1.at[pl.ds(0, szb)])

        def srow(rb, _, o=o):
            s16 = scale_v[pl.ds(o + rb * 16, 16)]
            for j in range(16):
                r = rb * 16 + j
                sc = s16[j]
                for g in range(4):
                    rows0[r, pl.ds(g * 16, 16)] = (
                        rows0[r, pl.ds(g * 16, 16)] * sc
                        + rows1[r, pl.ds(g * 16, 16)]
                    )
            return 0
        lax.fori_loop(0, szb // 16, srow, 0)
        pltpu.sync_copy(rows0.at[pl.ds(0, szb)], tout_hbm.at[pl.ds(g0 + o, szb)])


_sc_conv = pl.kernel(
    _sc_conv_body,
    out_type=jax.ShapeDtypeStruct((NPAD, C), jnp.float32),
    mesh=_MESH,
    compiler_params=pltpu.CompilerParams(needs_layout_passes=False, use_tc_tiling_on_sc=False),
    scratch_types=[
        pltpu.VMEM((CHMAX, CHUNK), jnp.int32),
        pltpu.VMEM((CHMAX, CHUNK), jnp.int32),
        pltpu.VMEM((CHUNK, C), jnp.float32),
        pltpu.VMEM((CHUNK, C), jnp.float32),
        pltpu.VMEM((TROWS,), jnp.float32),
        pltpu.VMEM((16,), jnp.int32),
        pltpu.VMEM_SHARED((NACC, C), jnp.float32),
        pltpu.VMEM_SHARED((NPAD, C), jnp.float32),
        pltpu.SemaphoreType.DMA,
        pltpu.SemaphoreType.DMA,
    ],
)


def kernel(x, edge_index, train_idx, labels, W1, b1, W2, b2):
    src = edge_index[0].astype(jnp.int32)
    dst = edge_index[1].astype(jnp.int32)
    tr = jnp.concatenate(
        [train_idx.astype(jnp.int32),
         jnp.full((NTR - train_idx.shape[0],), TRPAD, jnp.int32)])
    lab = jnp.concatenate(
        [labels.astype(jnp.int32),
         jnp.zeros((NTR - labels.shape[0],), jnp.int32)])

    p = _mlp_softmax(x, W1, b1, W2, b2)
    p_pad = jnp.pad(p, ((0, NPAD - N), (0, 0)))

    esrc, edst, nch, scale1, scale2, bias1, t0 = _sc_setup(
        src, dst, tr, lab, p_pad)
    esrc3 = esrc.reshape(NSC * NT, CHMAX, CHUNK)
    edst3 = edst.reshape(NSC * NT, CHMAX, CHUNK)

    t = t0
    for _ in range(DEPTH):
        t = _sc_conv(t, esrc3, edst3, nch, scale1, bias1)

    h0b, bias2 = _transition(p_pad, t)
    t = h0b
    for _ in range(DEPTH):
        t = _sc_conv(t, esrc3, edst3, nch, scale2, bias2)

    return _finalize(t[:N])


# speedup vs baseline: 1.1887x; 1.1887x over previous
import functools

import jax
import jax.numpy as jnp
from jax import lax
from jax.experimental import pallas as pl
from jax.experimental.pallas import tpu as pltpu
from jax.experimental.pallas import tpu_sc as plsc

N = 10000
E = 320000
FEATS = 128
HIDDEN = 64
C = 64
DEPTH = 10

NSC = 2
NT = 16
NPAD = 10240
ROWS_SC = NPAD // NSC
TROWS = ROWS_SC // NT
NACC = ROWS_SC + 128
ZR = NACC // NT

EPT = E // NT
STAGE = 2000
CHUNK = 128
CHMAX = (EPT + CHUNK - 1) // CHUNK + 1
EBUF = CHMAX * CHUNK

NTR = 1024
TPT = NTR // NT
TRPAD = 4 * NPAD


def _i16():
    return jnp.arange(16, dtype=jnp.int32)


def _mlp_body(x_ref, w1_ref, b1_ref, w2_ref, b2_ref, p_ref):
    h = jnp.maximum(x_ref[...] @ w1_ref[...] + b1_ref[...][None, :], 0.0)
    logits = h @ w2_ref[...] + b2_ref[...][None, :]
    m = jnp.max(logits, axis=1, keepdims=True)
    e = jnp.exp(logits - m)
    p_ref[...] = e / jnp.sum(e, axis=1, keepdims=True)


def _mlp_softmax(x, W1, b1, W2, b2):
    blk = 2000
    return pl.pallas_call(
        _mlp_body,
        grid=(N // blk,),
        in_specs=[
            pl.BlockSpec((blk, FEATS), lambda i: (i, 0)),
            pl.BlockSpec((FEATS, HIDDEN), lambda i: (0, 0)),
            pl.BlockSpec((HIDDEN,), lambda i: (0,)),
            pl.BlockSpec((HIDDEN, C), lambda i: (0, 0)),
            pl.BlockSpec((C,), lambda i: (0,)),
        ],
        out_specs=pl.BlockSpec((blk, C), lambda i: (i, 0)),
        out_shape=jax.ShapeDtypeStruct((N, C), jnp.float32),
    )(x, W1, b1, W2, b2)


def _trans_body(p_ref, t_ref, h0b_ref, b2_ref):
    h0b = p_ref[...] + t_ref[...]
    h0b_ref[...] = h0b
    b2_ref[...] = 0.1 * h0b


def _transition(p_pad, t10):
    blk = 2560
    return pl.pallas_call(
        _trans_body,
        grid=(NPAD // blk,),
        in_specs=[
            pl.BlockSpec((blk, C), lambda i: (i, 0)),
            pl.BlockSpec((blk, C), lambda i: (i, 0)),
        ],
        out_specs=[
            pl.BlockSpec((blk, C), lambda i: (i, 0)),
            pl.BlockSpec((blk, C), lambda i: (i, 0)),
        ],
        out_shape=[
            jax.ShapeDtypeStruct((NPAD, C), jnp.float32),
            jax.ShapeDtypeStruct((NPAD, C), jnp.float32),
        ],
    )(p_pad, t10)


def _log1p_body(t_ref, o_ref):
    o_ref[...] = jnp.log(t_ref[...] + 1.0)


def _finalize(t):
    blk = 2000
    return pl.pallas_call(
        _log1p_body,
        grid=(N // blk,),
        in_specs=[pl.BlockSpec((blk, C), lambda i: (i, 0))],
        out_specs=pl.BlockSpec((blk, C), lambda i: (i, 0)),
        out_shape=jax.ShapeDtypeStruct((N, C), jnp.float32),
    )(t)


_MESH = plsc.VectorSubcoreMesh(core_axis_name="c", subcore_axis_name="s")


def _sc_setup_body(
    src_hbm, dst_hbm, tr_hbm, lab_hbm, p_hbm,
    esrc_hbm, edst_hbm, nch_hbm, scale1_hbm, scale2_hbm, bias1_hbm, t0_hbm,
    stage_s, stage_d, out_src, out_dst, ones_e, zsmall, pblk, bblk,
    trows, tidx, tloc, tlab, ones64, deg_t, msk_t, s1_t, s2_t, n16,
    degsp, masksp, bias1sp, sem,
):
    c = lax.axis_index("c")
    s = lax.axis_index("s")
    w = c * NT + s
    lo = s * TROWS
    g0 = c * ROWS_SC + s * TROWS
    sc_lo = c * ROWS_SC

    z16 = jnp.zeros((16,), jnp.float32)
    o16 = jnp.ones((16,), jnp.float32)
    i16 = _i16()

    def fill_z(j, _):
        zsmall[pl.ds(j * 16, 16)] = z16
        return 0
    lax.fori_loop(0, 336 // 16, fill_z, 0)

    def fill_bblk(r, _):
        for g in range(4):
            bblk[r, pl.ds(g * 16, 16)] = z16
        return 0
    lax.fori_loop(0, 16, fill_bblk, 0)

    def fill_ones(j, _):
        ones_e[pl.ds(j * 16, 16)] = o16
        return 0
    lax.fori_loop(0, EBUF // 16, fill_ones, 0)

    for g in range(4):
        ones64[pl.ds(g * 16, 16)] = o16

    def fill_pad(j, _):
        lane = j * 16 + i16
        out_src[pl.ds(j * 16, 16)] = lane % N
        out_dst[pl.ds(j * 16, 16)] = ROWS_SC + (lane % 128)
        return 0
    lax.fori_loop(0, EBUF // 16, fill_pad, 0)

    pltpu.sync_copy(zsmall.at[pl.ds(0, ZR)], degsp.at[pl.ds(s * ZR, ZR)])
    pltpu.sync_copy(zsmall.at[pl.ds(0, ZR)], masksp.at[pl.ds(s * ZR, ZR)])

    def zero_b1(j, _):
        pltpu.sync_copy(bblk, bias1sp.at[pl.ds(s * ZR + j * 16, 16)])
        return 0
    lax.fori_loop(0, ZR // 16, zero_b1, 0)
    pltpu.sync_copy(bblk.at[pl.ds(0, ZR - 20 * 16)],
                    bias1sp.at[pl.ds(s * ZR + 20 * 16, ZR - 20 * 16)])
    plsc.subcore_barrier()

    def pass_body(k, cnt):
        pltpu.sync_copy(src_hbm.at[pl.ds(s * EPT + k * STAGE, STAGE)], stage_s)
        pltpu.sync_copy(dst_hbm.at[pl.ds(s * EPT + k * STAGE, STAGE)], stage_d)

        def grp(gi, cnt):
            sv = stage_s[pl.ds(gi * 16, 16)]
            dv = stage_d[pl.ds(gi * 16, 16)]
            m = (dv >= sc_lo) & (dv < sc_lo + ROWS_SC)
            mi = m.astype(jnp.int32)
            pos = cnt + plsc.cumsum(mi) - mi
            plsc.store_scatter(out_src, [pos], sv, mask=m)
            plsc.store_scatter(out_dst, [pos], dv - sc_lo, mask=m)
            npop = plsc.all_reduce_population_count(m)
            return cnt + npop[0]
        return lax.fori_loop(0, STAGE // 16, grp, cnt)

    cnt = lax.fori_loop(0, EPT // STAGE, pass_body, jnp.int32(0))
    nch = (cnt + CHUNK - 1) // CHUNK

    pltpu.sync_copy(ones_e, degsp.at[out_dst], add=True)

    pltpu.sync_copy(tr_hbm.at[pl.ds(s * TPT, TPT)], tidx)
    pltpu.sync_copy(lab_hbm.at[pl.ds(s * TPT, TPT)], tlab)
    for g in range(4):
        tv = tidx[pl.ds(g * 16, 16)]
        lv = tv - sc_lo
        valid = (lv >= 0) & (lv < ROWS_SC)
        spread = g * 16 + i16
        tloc[pl.ds(g * 16, 16)] = jnp.where(valid, lv, ROWS_SC + (spread % 128))
        tidx[pl.ds(g * 16, 16)] = jnp.where(tv < NPAD, tv, spread)

    pltpu.async_copy(p_hbm.at[tidx], trows, sem).wait()

    def neg_row(j, _):
        for g in range(4):
            trows[j, pl.ds(g * 16, 16)] = -trows[j, pl.ds(g * 16, 16)]
        return 0
    lax.fori_loop(0, TPT, neg_row, 0)

    for g in range(4):
        jv = g * 16 + i16
        lv16 = tlab[pl.ds(g * 16, 16)]
        plsc.addupdate_scatter(trows, [jv, lv16], o16)

    pltpu.sync_copy(ones64, masksp.at[tloc], add=True)
    pltpu.sync_copy(trows, bias1sp.at[tloc], add=True)
    plsc.subcore_barrier()

    pltpu.sync_copy(degsp.at[pl.ds(lo, TROWS)], deg_t)
    pltpu.sync_copy(masksp.at[pl.ds(lo, TROWS)], msk_t)

    def scales(j, _):
        d = deg_t[pl.ds(j * 16, 16)]
        mk = msk_t[pl.ds(j * 16, 16)]
        dinv = 1.0 / jnp.maximum(d, 1.0)
        s1_t[pl.ds(j * 16, 16)] = dinv * (1.0 - mk)
        s2_t[pl.ds(j * 16, 16)] = 0.9 * dinv
        return 0
    lax.fori_loop(0, TROWS // 16, scales, 0)

    pltpu.sync_copy(s1_t, scale1_hbm.at[pl.ds(g0, TROWS)])
    pltpu.sync_copy(s2_t, scale2_hbm.at[pl.ds(g0, TROWS)])

    pltpu.sync_copy(bias1sp.at[pl.ds(lo, TROWS)], bias1_hbm.at[pl.ds(g0, TROWS)])

    def t0_blk(rb, _):
        pltpu.sync_copy(p_hbm.at[pl.ds(g0 + rb * 16, 16)], pblk)
        pltpu.sync_copy(bias1sp.at[pl.ds(lo + rb * 16, 16)], bblk)
        m16 = msk_t[pl.ds(rb * 16, 16)]
        for j in range(16):
            sc0 = 1.0 - m16[j]
            for g in range(4):
                pblk[j, pl.ds(g * 16, 16)] = (
                    bblk[j, pl.ds(g * 16, 16)]
                    - pblk[j, pl.ds(g * 16, 16)] * sc0
                )
        pltpu.sync_copy(pblk, t0_hbm.at[pl.ds(g0 + rb * 16, 16)])
        return 0
    lax.fori_loop(0, TROWS // 16, t0_blk, 0)

    n16[...] = jnp.full((16,), nch, jnp.int32)
    pltpu.sync_copy(n16, nch_hbm.at[w])
    pltpu.sync_copy(out_src, esrc_hbm.at[w])
    pltpu.sync_copy(out_dst, edst_hbm.at[w])


_sc_setup = pl.kernel(
    _sc_setup_body,
    out_type=[
        jax.ShapeDtypeStruct((NSC * NT, EBUF), jnp.int32),
        jax.ShapeDtypeStruct((NSC * NT, EBUF), jnp.int32),
        jax.ShapeDtypeStruct((NSC * NT, 16), jnp.int32),
        jax.ShapeDtypeStruct((NPAD,), jnp.float32),
        jax.ShapeDtypeStruct((NPAD,), jnp.float32),
        jax.ShapeDtypeStruct((NPAD, C), jnp.float32),
        jax.ShapeDtypeStruct((NPAD, C), jnp.float32),
    ],
    mesh=_MESH,
    compiler_params=pltpu.CompilerParams(needs_layout_passes=False, use_tc_tiling_on_sc=False),
    scratch_types=[
        pltpu.VMEM((STAGE,), jnp.int32),
        pltpu.VMEM((STAGE,), jnp.int32),
        pltpu.VMEM((EBUF,), jnp.int32),
        pltpu.VMEM((EBUF,), jnp.int32),
        pltpu.VMEM((EBUF,), jnp.float32),
        pltpu.VMEM((336,), jnp.float32),
        pltpu.VMEM((16, C), jnp.float32),
        pltpu.VMEM((16, C), jnp.float32),
        pltpu.VMEM((TPT, C), jnp.float32),
        pltpu.VMEM((TPT,), jnp.int32),
        pltpu.VMEM((TPT,), jnp.int32),
        pltpu.VMEM((TPT,), jnp.int32),
        pltpu.VMEM((TPT,), jnp.float32),
        pltpu.VMEM((TROWS,), jnp.float32),
        pltpu.VMEM((TROWS,), jnp.float32),
        pltpu.VMEM((TROWS,), jnp.float32),
        pltpu.VMEM((TROWS,), jnp.float32),
        pltpu.VMEM((16,), jnp.int32),
        pltpu.VMEM_SHARED((NACC,), jnp.float32),
        pltpu.VMEM_SHARED((NACC,), jnp.float32),
        pltpu.VMEM_SHARED((NACC, C), jnp.float32),
        pltpu.SemaphoreType.DMA,
    ],
)


def _sc_conv_body(
    t_hbm, esrc_hbm, edst_hbm, nch_hbm, scale_hbm, bias_hbm,
    tout_hbm,
    esrc_v, edst_v, rows0, rows1, accbuf, bias_v, scale_v, n16,
    accsp, sem0, sem1,
):
    c = lax.axis_index("c")
    s = lax.axis_index("s")
    w = c * NT + s
    lo = s * TROWS
    g0 = c * ROWS_SC + s * TROWS

    z16 = jnp.zeros((16,), jnp.float32)

    def fill_acc(r, _):
        for g in range(4):
            accbuf[r, pl.ds(g * 16, 16)] = z16
        return 0
    lax.fori_loop(0, TROWS, fill_acc, 0)
    pltpu.sync_copy(accbuf, accsp.at[pl.ds(s * ZR, TROWS)])
    pltpu.sync_copy(accbuf.at[pl.ds(0, ZR - TROWS)],
                    accsp.at[pl.ds(s * ZR + TROWS, ZR - TROWS)])

    pltpu.sync_copy(esrc_hbm.at[w], esrc_v)
    pltpu.sync_copy(edst_hbm.at[w], edst_v)
    pltpu.sync_copy(nch_hbm.at[w], n16)
    pltpu.sync_copy(scale_hbm.at[pl.ds(g0, TROWS)], scale_v)
    pltpu.sync_copy(bias_hbm.at[pl.ds(g0, TROWS)], bias_v)
    nch = lax.reduce_max(n16[...], (0,))
    plsc.subcore_barrier()

    @pl.when(nch > 0)
    def _():
        pltpu.async_copy(t_hbm.at[esrc_v.at[0]], rows0, sem0)

    @pl.when(nch > 1)
    def _():
        pltpu.async_copy(t_hbm.at[esrc_v.at[1]], rows1, sem1)

    def pair(p2, _):
        c0 = p2 * 2
        c1 = c0 + 1

        @pl.when(c0 < nch)
        def _():
            pltpu.make_async_copy(t_hbm.at[esrc_v.at[c0]], rows0, sem0).wait()
            pltpu.sync_copy(rows0, accsp.at[edst_v.at[c0]], add=True)

            @pl.when(c0 + 2 < nch)
            def _():
                pltpu.async_copy(t_hbm.at[esrc_v.at[c0 + 2]], rows0, sem0)

        @pl.when(c1 < nch)
        def _():
            pltpu.make_async_copy(t_hbm.at[esrc_v.at[c1]], rows1, sem1).wait()
            pltpu.sync_copy(rows1, accsp.at[edst_v.at[c1]], add=True)

            @pl.when(c1 + 2 < nch)
            def _():
                pltpu.async_copy(t_hbm.at[esrc_v.at[c1 + 2]], rows1, sem1)

        return 0

    lax.fori_loop(0, (nch + 1) // 2, pair, 0)
    plsc.subcore_barrier()

    pltpu.sync_copy(accsp.at[pl.ds(lo, TROWS)], accbuf)

    def srow(rb, _):
        s16 = scale_v[pl.ds(rb * 16, 16)]
        for j in range(16):
            r = rb * 16 + j
            sc = s16[j]
            for g in range(4):
                accbuf[r, pl.ds(g * 16, 16)] = (
                    accbuf[r, pl.ds(g * 16, 16)] * sc
                    + bias_v[r, pl.ds(g * 16, 16)]
                )
        return 0
    lax.fori_loop(0, TROWS // 16, srow, 0)
    pltpu.sync_copy(accbuf, tout_hbm.at[pl.ds(g0, TROWS)])


_sc_conv = pl.kernel(
    _sc_conv_body,
    out_type=jax.ShapeDtypeStruct((NPAD, C), jnp.float32),
    mesh=_MESH,
    compiler_params=pltpu.CompilerParams(needs_layout_passes=False, use_tc_tiling_on_sc=False),
    scratch_types=[
        pltpu.VMEM((CHMAX, CHUNK), jnp.int32),
        pltpu.VMEM((CHMAX, CHUNK), jnp.int32),
        pltpu.VMEM((CHUNK, C), jnp.float32),
        pltpu.VMEM((CHUNK, C), jnp.float32),
        pltpu.VMEM((TROWS, C), jnp.float32),
        pltpu.VMEM((TROWS, C), jnp.float32),
        pltpu.VMEM((TROWS,), jnp.float32),
        pltpu.VMEM((16,), jnp.int32),
        pltpu.VMEM_SHARED((NACC, C), jnp.float32),
        pltpu.SemaphoreType.DMA,
        pltpu.SemaphoreType.DMA,
    ],
)


def kernel(x, edge_index, train_idx, labels, W1, b1, W2, b2):
    src = edge_index[0].astype(jnp.int32)
    dst = edge_index[1].astype(jnp.int32)
    tr = jnp.concatenate(
        [train_idx.astype(jnp.int32),
         jnp.full((NTR - train_idx.shape[0],), TRPAD, jnp.int32)])
    lab = jnp.concatenate(
        [labels.astype(jnp.int32),
         jnp.zeros((NTR - labels.shape[0],), jnp.int32)])

    p = _mlp_softmax(x, W1, b1, W2, b2)
    p_pad = jnp.pad(p, ((0, NPAD - N), (0, 0)))

    esrc, edst, nch, scale1, scale2, bias1, t0 = _sc_setup(
        src, dst, tr, lab, p_pad)
    esrc3 = esrc.reshape(NSC * NT, CHMAX, CHUNK)
    edst3 = edst.reshape(NSC * NT, CHMAX, CHUNK)

    t = t0
    for _ in range(DEPTH):
        t = _sc_conv(t, esrc3, edst3, nch, scale1, bias1)

    h0b, bias2 = _transition(p_pad, t)
    t = h0b
    for _ in range(DEPTH):
        t = _sc_conv(t, esrc3, edst3, nch, scale2, bias2)

    return _finalize(t[:N])


# speedup vs baseline: 1.3735x; 1.1554x over previous
import functools

import jax
import jax.numpy as jnp
from jax import lax
from jax.experimental import pallas as pl
from jax.experimental.pallas import tpu as pltpu
from jax.experimental.pallas import tpu_sc as plsc

N = 10000
E = 320000
FEATS = 128
HIDDEN = 64
C = 64
DEPTH = 10

NSC = 2
NT = 16
NPAD = 10240
ROWS_SC = NPAD // NSC
TROWS = ROWS_SC // NT
NACC = ROWS_SC + 128
ZR = NACC // NT

EPT = E // NT
STAGE = 2000
CHUNK = 128
CHMAX = (EPT + CHUNK - 1) // CHUNK + 1
EBUF = CHMAX * CHUNK

NTR = 1024
TPT = NTR // NT
TRPAD = 4 * NPAD


def _i16():
    return jnp.arange(16, dtype=jnp.int32)


def _mlp_body(x_ref, w1_ref, b1_ref, w2_ref, b2_ref, p_ref):
    h = jnp.maximum(x_ref[...] @ w1_ref[...] + b1_ref[...][None, :], 0.0)
    logits = h @ w2_ref[...] + b2_ref[...][None, :]
    m = jnp.max(logits, axis=1, keepdims=True)
    e = jnp.exp(logits - m)
    p_ref[...] = e / jnp.sum(e, axis=1, keepdims=True)


def _mlp_softmax(x, W1, b1, W2, b2):
    blk = 2000
    return pl.pallas_call(
        _mlp_body,
        grid=(N // blk,),
        in_specs=[
            pl.BlockSpec((blk, FEATS), lambda i: (i, 0)),
            pl.BlockSpec((FEATS, HIDDEN), lambda i: (0, 0)),
            pl.BlockSpec((HIDDEN,), lambda i: (0,)),
            pl.BlockSpec((HIDDEN, C), lambda i: (0, 0)),
            pl.BlockSpec((C,), lambda i: (0,)),
        ],
        out_specs=pl.BlockSpec((blk, C), lambda i: (i, 0)),
        out_shape=jax.ShapeDtypeStruct((N, C), jnp.float32),
    )(x, W1, b1, W2, b2)


def _trans_body(p_ref, t_ref, h0b_ref, b2_ref):
    h0b = p_ref[...] + t_ref[...]
    h0b_ref[...] = h0b
    b2_ref[...] = 0.1 * h0b


def _transition(p_pad, t10):
    blk = 2560
    return pl.pallas_call(
        _trans_body,
        grid=(NPAD // blk,),
        in_specs=[
            pl.BlockSpec((blk, C), lambda i: (i, 0)),
            pl.BlockSpec((blk, C), lambda i: (i, 0)),
        ],
        out_specs=[
            pl.BlockSpec((blk, C), lambda i: (i, 0)),
            pl.BlockSpec((blk, C), lambda i: (i, 0)),
        ],
        out_shape=[
            jax.ShapeDtypeStruct((NPAD, C), jnp.float32),
            jax.ShapeDtypeStruct((NPAD, C), jnp.float32),
        ],
    )(p_pad, t10)


def _log1p_body(t_ref, o_ref):
    o_ref[...] = jnp.log(t_ref[...] + 1.0)


def _finalize(t):
    blk = 2000
    return pl.pallas_call(
        _log1p_body,
        grid=(N // blk,),
        in_specs=[pl.BlockSpec((blk, C), lambda i: (i, 0))],
        out_specs=pl.BlockSpec((blk, C), lambda i: (i, 0)),
        out_shape=jax.ShapeDtypeStruct((N, C), jnp.float32),
    )(t)


_MESH = plsc.VectorSubcoreMesh(core_axis_name="c", subcore_axis_name="s")


def _sc_setup_body(
    src_hbm, dst_hbm, tr_hbm, lab_hbm, p_hbm,
    esrc_hbm, edst_hbm, nch_hbm, scale1_hbm, scale2_hbm, bias1_hbm, t0_hbm,
    stage_s, stage_d, out_src, out_dst, ones_e, zsmall, pblk, bblk,
    trows, tidx, tloc, tlab, ones64, deg_t, msk_t, s1_t, s2_t, n16,
    degsp, masksp, bias1sp, sem,
):
    c = lax.axis_index("c")
    s = lax.axis_index("s")
    w = c * NT + s
    lo = s * TROWS
    g0 = c * ROWS_SC + s * TROWS
    sc_lo = c * ROWS_SC

    z16 = jnp.zeros((16,), jnp.float32)
    o16 = jnp.ones((16,), jnp.float32)
    i16 = _i16()

    def fill_z(j, _):
        zsmall[pl.ds(j * 16, 16)] = z16
        return 0
    lax.fori_loop(0, 336 // 16, fill_z, 0)

    def fill_bblk(r, _):
        for g in range(4):
            bblk[r, pl.ds(g * 16, 16)] = z16
        return 0
    lax.fori_loop(0, 16, fill_bblk, 0)

    def fill_ones(j, _):
        ones_e[pl.ds(j * 16, 16)] = o16
        return 0
    lax.fori_loop(0, EBUF // 16, fill_ones, 0)

    for g in range(4):
        ones64[pl.ds(g * 16, 16)] = o16

    def fill_pad(j, _):
        lane = j * 16 + i16
        out_src[pl.ds(j * 16, 16)] = lane % N
        out_dst[pl.ds(j * 16, 16)] = ROWS_SC + (lane % 128)
        return 0
    lax.fori_loop(0, EBUF // 16, fill_pad, 0)

    pltpu.sync_copy(zsmall.at[pl.ds(0, ZR)], degsp.at[pl.ds(s * ZR, ZR)])
    pltpu.sync_copy(zsmall.at[pl.ds(0, ZR)], masksp.at[pl.ds(s * ZR, ZR)])

    def zero_b1(j, _):
        pltpu.sync_copy(bblk, bias1sp.at[pl.ds(s * ZR + j * 16, 16)])
        return 0
    lax.fori_loop(0, ZR // 16, zero_b1, 0)
    pltpu.sync_copy(bblk.at[pl.ds(0, ZR - 20 * 16)],
                    bias1sp.at[pl.ds(s * ZR + 20 * 16, ZR - 20 * 16)])
    plsc.subcore_barrier()

    def pass_body(k, cnt):
        pltpu.sync_copy(src_hbm.at[pl.ds(s * EPT + k * STAGE, STAGE)], stage_s)
        pltpu.sync_copy(dst_hbm.at[pl.ds(s * EPT + k * STAGE, STAGE)], stage_d)

        def grp(gi, cnt):
            sv = stage_s[pl.ds(gi * 16, 16)]
            dv = stage_d[pl.ds(gi * 16, 16)]
            m = (dv >= sc_lo) & (dv < sc_lo + ROWS_SC)
            mi = m.astype(jnp.int32)
            pos = cnt + plsc.cumsum(mi) - mi
            plsc.store_scatter(out_src, [pos], sv, mask=m)
            plsc.store_scatter(out_dst, [pos], dv - sc_lo, mask=m)
            npop = plsc.all_reduce_population_count(m)
            return cnt + npop[0]
        return lax.fori_loop(0, STAGE // 16, grp, cnt)

    cnt = lax.fori_loop(0, EPT // STAGE, pass_body, jnp.int32(0))
    nch = (cnt + CHUNK - 1) // CHUNK

    pltpu.sync_copy(ones_e, degsp.at[out_dst], add=True)

    pltpu.sync_copy(tr_hbm.at[pl.ds(s * TPT, TPT)], tidx)
    pltpu.sync_copy(lab_hbm.at[pl.ds(s * TPT, TPT)], tlab)
    for g in range(4):
        tv = tidx[pl.ds(g * 16, 16)]
        lv = tv - sc_lo
        valid = (lv >= 0) & (lv < ROWS_SC)
        spread = g * 16 + i16
        tloc[pl.ds(g * 16, 16)] = jnp.where(valid, lv, ROWS_SC + (spread % 128))
        tidx[pl.ds(g * 16, 16)] = jnp.where(tv < NPAD, tv, spread)

    pltpu.async_copy(p_hbm.at[tidx], trows, sem).wait()

    def neg_row(j, _):
        for g in range(4):
            trows[j, pl.ds(g * 16, 16)] = -trows[j, pl.ds(g * 16, 16)]
        return 0
    lax.fori_loop(0, TPT, neg_row, 0)

    for g in range(4):
        jv = g * 16 + i16
        lv16 = tlab[pl.ds(g * 16, 16)]
        plsc.addupdate_scatter(trows, [jv, lv16], o16)

    pltpu.sync_copy(ones64, masksp.at[tloc], add=True)
    pltpu.sync_copy(trows, bias1sp.at[tloc], add=True)
    plsc.subcore_barrier()

    pltpu.sync_copy(degsp.at[pl.ds(lo, TROWS)], deg_t)
    pltpu.sync_copy(masksp.at[pl.ds(lo, TROWS)], msk_t)

    def scales(j, _):
        d = deg_t[pl.ds(j * 16, 16)]
        mk = msk_t[pl.ds(j * 16, 16)]
        dinv = 1.0 / jnp.maximum(d, 1.0)
        s1_t[pl.ds(j * 16, 16)] = dinv * (1.0 - mk)
        s2_t[pl.ds(j * 16, 16)] = 0.9 * dinv
        return 0
    lax.fori_loop(0, TROWS // 16, scales, 0)

    pltpu.sync_copy(s1_t, scale1_hbm.at[pl.ds(g0, TROWS)])
    pltpu.sync_copy(s2_t, scale2_hbm.at[pl.ds(g0, TROWS)])

    pltpu.sync_copy(bias1sp.at[pl.ds(lo, TROWS)], bias1_hbm.at[pl.ds(g0, TROWS)])

    def t0_blk(rb, _):
        pltpu.sync_copy(p_hbm.at[pl.ds(g0 + rb * 16, 16)], pblk)
        pltpu.sync_copy(bias1sp.at[pl.ds(lo + rb * 16, 16)], bblk)
        m16 = msk_t[pl.ds(rb * 16, 16)]
        for j in range(16):
            sc0 = 1.0 - m16[j]
            for g in range(4):
                pblk[j, pl.ds(g * 16, 16)] = (
                    bblk[j, pl.ds(g * 16, 16)]
                    - pblk[j, pl.ds(g * 16, 16)] * sc0
                )
        pltpu.sync_copy(pblk, t0_hbm.at[pl.ds(g0 + rb * 16, 16)])
        return 0
    lax.fori_loop(0, TROWS // 16, t0_blk, 0)

    n16[...] = jnp.full((16,), nch, jnp.int32)
    pltpu.sync_copy(n16, nch_hbm.at[w])
    pltpu.sync_copy(out_src, esrc_hbm.at[w])
    pltpu.sync_copy(out_dst, edst_hbm.at[w])


_sc_setup = pl.kernel(
    _sc_setup_body,
    out_type=[
        jax.ShapeDtypeStruct((NSC * NT, EBUF), jnp.int32),
        jax.ShapeDtypeStruct((NSC * NT, EBUF), jnp.int32),
        jax.ShapeDtypeStruct((NSC * NT, 16), jnp.int32),
        jax.ShapeDtypeStruct((NPAD,), jnp.float32),
        jax.ShapeDtypeStruct((NPAD,), jnp.float32),
        jax.ShapeDtypeStruct((NPAD, C), jnp.float32),
        jax.ShapeDtypeStruct((NPAD, C), jnp.float32),
    ],
    mesh=_MESH,
    compiler_params=pltpu.CompilerParams(needs_layout_passes=False, use_tc_tiling_on_sc=False),
    scratch_types=[
        pltpu.VMEM((STAGE,), jnp.int32),
        pltpu.VMEM((STAGE,), jnp.int32),
        pltpu.VMEM((EBUF,), jnp.int32),
        pltpu.VMEM((EBUF,), jnp.int32),
        pltpu.VMEM((EBUF,), jnp.float32),
        pltpu.VMEM((336,), jnp.float32),
        pltpu.VMEM((16, C), jnp.float32),
        pltpu.VMEM((16, C), jnp.float32),
        pltpu.VMEM((TPT, C), jnp.float32),
        pltpu.VMEM((TPT,), jnp.int32),
        pltpu.VMEM((TPT,), jnp.int32),
        pltpu.VMEM((TPT,), jnp.int32),
        pltpu.VMEM((TPT,), jnp.float32),
        pltpu.VMEM((TROWS,), jnp.float32),
        pltpu.VMEM((TROWS,), jnp.float32),
        pltpu.VMEM((TROWS,), jnp.float32),
        pltpu.VMEM((TROWS,), jnp.float32),
        pltpu.VMEM((16,), jnp.int32),
        pltpu.VMEM_SHARED((NACC,), jnp.float32),
        pltpu.VMEM_SHARED((NACC,), jnp.float32),
        pltpu.VMEM_SHARED((NACC, C), jnp.float32),
        pltpu.SemaphoreType.DMA,
    ],
)


def _sc_conv_body(
    t_hbm, esrc_hbm, edst_hbm, nch_hbm, scale_hbm, bias_hbm,
    tout_hbm,
    esrc_v, edst_v, rows0, rows1, rows2, accbuf, bias_v, scale_v, n16,
    accsp, sem0, sem1, sem2,
):
    c = lax.axis_index("c")
    s = lax.axis_index("s")
    w = c * NT + s
    lo = s * TROWS
    g0 = c * ROWS_SC + s * TROWS

    z16 = jnp.zeros((16,), jnp.float32)

    def fill_acc(r, _):
        for g in range(4):
            accbuf[r, pl.ds(g * 16, 16)] = z16
        return 0
    lax.fori_loop(0, TROWS, fill_acc, 0)
    pltpu.sync_copy(accbuf, accsp.at[pl.ds(s * ZR, TROWS)])
    pltpu.sync_copy(accbuf.at[pl.ds(0, ZR - TROWS)],
                    accsp.at[pl.ds(s * ZR + TROWS, ZR - TROWS)])

    pltpu.sync_copy(esrc_hbm.at[w], esrc_v)
    pltpu.sync_copy(edst_hbm.at[w], edst_v)
    pltpu.sync_copy(nch_hbm.at[w], n16)
    pltpu.sync_copy(scale_hbm.at[pl.ds(g0, TROWS)], scale_v)
    pltpu.sync_copy(bias_hbm.at[pl.ds(g0, TROWS)], bias_v)
    nch = lax.reduce_max(n16[...], (0,))
    plsc.subcore_barrier()

    rows = (rows0, rows1, rows2)
    sems = (sem0, sem1, sem2)

    @pl.when(nch > 0)
    def _():
        pltpu.async_copy(t_hbm.at[esrc_v.at[0]], rows0, sem0)

    @pl.when(nch > 1)
    def _():
        pltpu.async_copy(t_hbm.at[esrc_v.at[1]], rows1, sem1)

    def triple(q, _):
        for j in range(3):
            cc = q * 3 + j
            jn = (j + 2) % 3

            @pl.when(cc < nch)
            def _(j=j, jn=jn, cc=cc):
                pltpu.make_async_copy(
                    t_hbm.at[esrc_v.at[cc]], rows[j], sems[j]).wait()

                @pl.when(cc + 2 < nch)
                def _():
                    pltpu.async_copy(
                        t_hbm.at[esrc_v.at[cc + 2]], rows[jn], sems[jn])

                pltpu.sync_copy(rows[j], accsp.at[edst_v.at[cc]], add=True)
        return 0

    lax.fori_loop(0, (nch + 2) // 3, triple, 0)
    plsc.subcore_barrier()

    pltpu.sync_copy(accsp.at[pl.ds(lo, TROWS)], accbuf)

    def srow(rb, _):
        s16 = scale_v[pl.ds(rb * 16, 16)]
        for j in range(16):
            r = rb * 16 + j
            sc = s16[j]
            for g in range(4):
                accbuf[r, pl.ds(g * 16, 16)] = (
                    accbuf[r, pl.ds(g * 16, 16)] * sc
                    + bias_v[r, pl.ds(g * 16, 16)]
                )
        return 0
    lax.fori_loop(0, TROWS // 16, srow, 0)
    pltpu.sync_copy(accbuf, tout_hbm.at[pl.ds(g0, TROWS)])


_sc_conv = pl.kernel(
    _sc_conv_body,
    out_type=jax.ShapeDtypeStruct((NPAD, C), jnp.float32),
    mesh=_MESH,
    compiler_params=pltpu.CompilerParams(needs_layout_passes=False, use_tc_tiling_on_sc=False),
    scratch_types=[
        pltpu.VMEM((CHMAX, CHUNK), jnp.int32),
        pltpu.VMEM((CHMAX, CHUNK), jnp.int32),
        pltpu.VMEM((CHUNK, C), jnp.float32),
        pltpu.VMEM((CHUNK, C), jnp.float32),
        pltpu.VMEM((CHUNK, C), jnp.float32),
        pltpu.VMEM((TROWS, C), jnp.float32),
        pltpu.VMEM((TROWS, C), jnp.float32),
        pltpu.VMEM((TROWS,), jnp.float32),
        pltpu.VMEM((16,), jnp.int32),
        pltpu.VMEM_SHARED((NACC, C), jnp.float32),
        pltpu.SemaphoreType.DMA,
        pltpu.SemaphoreType.DMA,
        pltpu.SemaphoreType.DMA,
    ],
)


def kernel(x, edge_index, train_idx, labels, W1, b1, W2, b2):
    src = edge_index[0].astype(jnp.int32)
    dst = edge_index[1].astype(jnp.int32)
    tr = jnp.concatenate(
        [train_idx.astype(jnp.int32),
         jnp.full((NTR - train_idx.shape[0],), TRPAD, jnp.int32)])
    lab = jnp.concatenate(
        [labels.astype(jnp.int32),
         jnp.zeros((NTR - labels.shape[0],), jnp.int32)])

    p = _mlp_softmax(x, W1, b1, W2, b2)
    p_pad = jnp.pad(p, ((0, NPAD - N), (0, 0)))

    esrc, edst, nch, scale1, scale2, bias1, t0 = _sc_setup(
        src, dst, tr, lab, p_pad)
    esrc3 = esrc.reshape(NSC * NT, CHMAX, CHUNK)
    edst3 = edst.reshape(NSC * NT, CHMAX, CHUNK)

    t = t0
    for _ in range(DEPTH):
        t = _sc_conv(t, esrc3, edst3, nch, scale1, bias1)

    h0b, bias2 = _transition(p_pad, t)
    t = h0b
    for _ in range(DEPTH):
        t = _sc_conv(t, esrc3, edst3, nch, scale2, bias2)

    return _finalize(t[:N])


# speedup vs baseline: 1.4252x; 1.0376x over previous
import functools

import jax
import jax.numpy as jnp
from jax import lax
from jax.experimental import pallas as pl
from jax.experimental.pallas import tpu as pltpu
from jax.experimental.pallas import tpu_sc as plsc

N = 10000
E = 320000
FEATS = 128
HIDDEN = 64
C = 64
DEPTH = 10

NSC = 2
NT = 16
NPAD = 10240
ROWS_SC = NPAD // NSC
TROWS = ROWS_SC // NT
NACC = ROWS_SC + 128
ZR = NACC // NT

EPT = E // NT
STAGE = 2000
CHUNK = 128
CHMAX = (EPT + CHUNK - 1) // CHUNK + 1
EBUF = CHMAX * CHUNK

NTR = 1024
TPT = NTR // NT
TRPAD = 4 * NPAD


def _i16():
    return jnp.arange(16, dtype=jnp.int32)


def _mlp_body(x_ref, w1_ref, b1_ref, w2_ref, b2_ref, p_ref):
    h = jnp.maximum(x_ref[...] @ w1_ref[...] + b1_ref[...][None, :], 0.0)
    logits = h @ w2_ref[...] + b2_ref[...][None, :]
    m = jnp.max(logits, axis=1, keepdims=True)
    e = jnp.exp(logits - m)
    p_ref[...] = e / jnp.sum(e, axis=1, keepdims=True)


def _mlp_softmax(x, W1, b1, W2, b2):
    blk = 2000
    return pl.pallas_call(
        _mlp_body,
        grid=(N // blk,),
        in_specs=[
            pl.BlockSpec((blk, FEATS), lambda i: (i, 0)),
            pl.BlockSpec((FEATS, HIDDEN), lambda i: (0, 0)),
            pl.BlockSpec((HIDDEN,), lambda i: (0,)),
            pl.BlockSpec((HIDDEN, C), lambda i: (0, 0)),
            pl.BlockSpec((C,), lambda i: (0,)),
        ],
        out_specs=pl.BlockSpec((blk, C), lambda i: (i, 0)),
        out_shape=jax.ShapeDtypeStruct((N, C), jnp.float32),
    )(x, W1, b1, W2, b2)


def _trans_body(p_ref, t_ref, h0b_ref, b2_ref):
    h0b = p_ref[...] + t_ref[...]
    h0b_ref[...] = h0b
    b2_ref[...] = 0.1 * h0b


def _transition(p_pad, t10):
    blk = 2560
    return pl.pallas_call(
        _trans_body,
        grid=(NPAD // blk,),
        in_specs=[
            pl.BlockSpec((blk, C), lambda i: (i, 0)),
            pl.BlockSpec((blk, C), lambda i: (i, 0)),
        ],
        out_specs=[
            pl.BlockSpec((blk, C), lambda i: (i, 0)),
            pl.BlockSpec((blk, C), lambda i: (i, 0)),
        ],
        out_shape=[
            jax.ShapeDtypeStruct((NPAD, C), jnp.float32),
            jax.ShapeDtypeStruct((NPAD, C), jnp.float32),
        ],
    )(p_pad, t10)


def _log1p_body(t_ref, o_ref):
    o_ref[...] = jnp.log(t_ref[...] + 1.0)


def _finalize(t):
    blk = 2000
    return pl.pallas_call(
        _log1p_body,
        grid=(N // blk,),
        in_specs=[pl.BlockSpec((blk, C), lambda i: (i, 0))],
        out_specs=pl.BlockSpec((blk, C), lambda i: (i, 0)),
        out_shape=jax.ShapeDtypeStruct((N, C), jnp.float32),
    )(t)


_MESH = plsc.VectorSubcoreMesh(core_axis_name="c", subcore_axis_name="s")


def _sc_setup_body(
    src_hbm, dst_hbm, tr_hbm, lab_hbm, p_hbm,
    esrc_hbm, edst_hbm, nch_hbm, scale1_hbm, scale2_hbm, bias1_hbm, t0_hbm,
    stage_s, stage_d, out_src, out_dst, ones_e, zsmall, pblk, bblk,
    trows, tidx, tloc, tlab, ones64, deg_t, msk_t, s1_t, s2_t, n16,
    degsp, masksp, bias1sp, sem,
):
    c = lax.axis_index("c")
    s = lax.axis_index("s")
    w = c * NT + s
    lo = s * TROWS
    g0 = c * ROWS_SC + s * TROWS
    sc_lo = c * ROWS_SC

    z16 = jnp.zeros((16,), jnp.float32)
    o16 = jnp.ones((16,), jnp.float32)
    i16 = _i16()

    def fill_z(j, _):
        zsmall[pl.ds(j * 16, 16)] = z16
        return 0
    lax.fori_loop(0, 336 // 16, fill_z, 0)

    def fill_bblk(r, _):
        for g in range(4):
            bblk[r, pl.ds(g * 16, 16)] = z16
        return 0
    lax.fori_loop(0, 16, fill_bblk, 0)

    def fill_ones(j, _):
        ones_e[pl.ds(j * 16, 16)] = o16
        return 0
    lax.fori_loop(0, EBUF // 16, fill_ones, 0)

    for g in range(4):
        ones64[pl.ds(g * 16, 16)] = o16

    def fill_pad(j, _):
        lane = j * 16 + i16
        out_src[pl.ds(j * 16, 16)] = lane % N
        out_dst[pl.ds(j * 16, 16)] = ROWS_SC + (lane % 128)
        return 0
    lax.fori_loop(0, EBUF // 16, fill_pad, 0)

    pltpu.sync_copy(zsmall.at[pl.ds(0, ZR)], degsp.at[pl.ds(s * ZR, ZR)])
    pltpu.sync_copy(zsmall.at[pl.ds(0, ZR)], masksp.at[pl.ds(s * ZR, ZR)])

    def zero_b1(j, _):
        pltpu.sync_copy(bblk, bias1sp.at[pl.ds(s * ZR + j * 16, 16)])
        return 0
    lax.fori_loop(0, ZR // 16, zero_b1, 0)
    pltpu.sync_copy(bblk.at[pl.ds(0, ZR - 20 * 16)],
                    bias1sp.at[pl.ds(s * ZR + 20 * 16, ZR - 20 * 16)])
    plsc.subcore_barrier()

    def pass_body(k, cnt):
        pltpu.sync_copy(src_hbm.at[pl.ds(s * EPT + k * STAGE, STAGE)], stage_s)
        pltpu.sync_copy(dst_hbm.at[pl.ds(s * EPT + k * STAGE, STAGE)], stage_d)

        def grp(gi, cnt):
            sv = stage_s[pl.ds(gi * 16, 16)]
            dv = stage_d[pl.ds(gi * 16, 16)]
            m = (dv >= sc_lo) & (dv < sc_lo + ROWS_SC)
            mi = m.astype(jnp.int32)
            pos = cnt + plsc.cumsum(mi) - mi
            plsc.store_scatter(out_src, [pos], sv, mask=m)
            plsc.store_scatter(out_dst, [pos], dv - sc_lo, mask=m)
            npop = plsc.all_reduce_population_count(m)
            return cnt + npop[0]
        return lax.fori_loop(0, STAGE // 16, grp, cnt)

    cnt = lax.fori_loop(0, EPT // STAGE, pass_body, jnp.int32(0))
    nch = (cnt + CHUNK - 1) // CHUNK

    pltpu.sync_copy(ones_e, degsp.at[out_dst], add=True)

    pltpu.sync_copy(tr_hbm.at[pl.ds(s * TPT, TPT)], tidx)
    pltpu.sync_copy(lab_hbm.at[pl.ds(s * TPT, TPT)], tlab)
    for g in range(4):
        tv = tidx[pl.ds(g * 16, 16)]
        lv = tv - sc_lo
        valid = (lv >= 0) & (lv < ROWS_SC)
        spread = g * 16 + i16
        tloc[pl.ds(g * 16, 16)] = jnp.where(valid, lv, ROWS_SC + (spread % 128))
        tidx[pl.ds(g * 16, 16)] = jnp.where(tv < NPAD, tv, spread)

    pltpu.async_copy(p_hbm.at[tidx], trows, sem).wait()

    def neg_row(j, _):
        for g in range(4):
            trows[j, pl.ds(g * 16, 16)] = -trows[j, pl.ds(g * 16, 16)]
        return 0
    lax.fori_loop(0, TPT, neg_row, 0)

    for g in range(4):
        jv = g * 16 + i16
        lv16 = tlab[pl.ds(g * 16, 16)]
        plsc.addupdate_scatter(trows, [jv, lv16], o16)

    pltpu.sync_copy(ones64, masksp.at[tloc], add=True)
    pltpu.sync_copy(trows, bias1sp.at[tloc], add=True)
    plsc.subcore_barrier()

    pltpu.sync_copy(degsp.at[pl.ds(lo, TROWS)], deg_t)
    pltpu.sync_copy(masksp.at[pl.ds(lo, TROWS)], msk_t)

    def scales(j, _):
        d = deg_t[pl.ds(j * 16, 16)]
        mk = msk_t[pl.ds(j * 16, 16)]
        dinv = 1.0 / jnp.maximum(d, 1.0)
        s1_t[pl.ds(j * 16, 16)] = dinv * (1.0 - mk)
        s2_t[pl.ds(j * 16, 16)] = 0.9 * dinv
        return 0
    lax.fori_loop(0, TROWS // 16, scales, 0)

    pltpu.sync_copy(s1_t, scale1_hbm.at[pl.ds(g0, TROWS)])
    pltpu.sync_copy(s2_t, scale2_hbm.at[pl.ds(g0, TROWS)])

    pltpu.sync_copy(bias1sp.at[pl.ds(lo, TROWS)], bias1_hbm.at[pl.ds(g0, TROWS)])

    def t0_blk(rb, _):
        pltpu.sync_copy(p_hbm.at[pl.ds(g0 + rb * 16, 16)], pblk)
        pltpu.sync_copy(bias1sp.at[pl.ds(lo + rb * 16, 16)], bblk)
        m16 = msk_t[pl.ds(rb * 16, 16)]
        for j in range(16):
            sc0 = 1.0 - m16[j]
            for g in range(4):
                pblk[j, pl.ds(g * 16, 16)] = (
                    bblk[j, pl.ds(g * 16, 16)]
                    - pblk[j, pl.ds(g * 16, 16)] * sc0
                )
        pltpu.sync_copy(pblk, t0_hbm.at[pl.ds(g0 + rb * 16, 16)])
        return 0
    lax.fori_loop(0, TROWS // 16, t0_blk, 0)

    n16[...] = jnp.full((16,), nch, jnp.int32)
    pltpu.sync_copy(n16, nch_hbm.at[w])
    pltpu.sync_copy(out_src, esrc_hbm.at[w])
    pltpu.sync_copy(out_dst, edst_hbm.at[w])


_sc_setup = pl.kernel(
    _sc_setup_body,
    out_type=[
        jax.ShapeDtypeStruct((NSC * NT, EBUF), jnp.int32),
        jax.ShapeDtypeStruct((NSC * NT, EBUF), jnp.int32),
        jax.ShapeDtypeStruct((NSC * NT, 16), jnp.int32),
        jax.ShapeDtypeStruct((NPAD,), jnp.float32),
        jax.ShapeDtypeStruct((NPAD,), jnp.float32),
        jax.ShapeDtypeStruct((NPAD, C), jnp.float32),
        jax.ShapeDtypeStruct((NPAD, C), jnp.float32),
    ],
    mesh=_MESH,
    compiler_params=pltpu.CompilerParams(needs_layout_passes=False, use_tc_tiling_on_sc=False),
    scratch_types=[
        pltpu.VMEM((STAGE,), jnp.int32),
        pltpu.VMEM((STAGE,), jnp.int32),
        pltpu.VMEM((EBUF,), jnp.int32),
        pltpu.VMEM((EBUF,), jnp.int32),
        pltpu.VMEM((EBUF,), jnp.float32),
        pltpu.VMEM((336,), jnp.float32),
        pltpu.VMEM((16, C), jnp.float32),
        pltpu.VMEM((16, C), jnp.float32),
        pltpu.VMEM((TPT, C), jnp.float32),
        pltpu.VMEM((TPT,), jnp.int32),
        pltpu.VMEM((TPT,), jnp.int32),
        pltpu.VMEM((TPT,), jnp.int32),
        pltpu.VMEM((TPT,), jnp.float32),
        pltpu.VMEM((TROWS,), jnp.float32),
        pltpu.VMEM((TROWS,), jnp.float32),
        pltpu.VMEM((TROWS,), jnp.float32),
        pltpu.VMEM((TROWS,), jnp.float32),
        pltpu.VMEM((16,), jnp.int32),
        pltpu.VMEM_SHARED((NACC,), jnp.float32),
        pltpu.VMEM_SHARED((NACC,), jnp.float32),
        pltpu.VMEM_SHARED((NACC, C), jnp.float32),
        pltpu.SemaphoreType.DMA,
    ],
)


def _sc_conv_body(
    t_hbm, esrc_hbm, edst_hbm, nch_hbm, scale_hbm, bias_hbm,
    tout_hbm,
    esrc_v, edst_v, rows0, rows1, rows2, accbuf, bias_v, scale_v, n16,
    accsp, sem0, sem1, sem2,
):
    c = lax.axis_index("c")
    s = lax.axis_index("s")
    w = c * NT + s
    lo = s * TROWS
    g0 = c * ROWS_SC + s * TROWS

    z16 = jnp.zeros((16,), jnp.float32)

    def fill_acc(r, _):
        for g in range(4):
            accbuf[r, pl.ds(g * 16, 16)] = z16
        return 0
    lax.fori_loop(0, TROWS, fill_acc, 0)
    pltpu.sync_copy(accbuf, accsp.at[pl.ds(s * ZR, TROWS)])
    pltpu.sync_copy(accbuf.at[pl.ds(0, ZR - TROWS)],
                    accsp.at[pl.ds(s * ZR + TROWS, ZR - TROWS)])

    pltpu.sync_copy(esrc_hbm.at[w], esrc_v)
    pltpu.sync_copy(edst_hbm.at[w], edst_v)
    pltpu.sync_copy(nch_hbm.at[w], n16)
    pltpu.sync_copy(scale_hbm.at[pl.ds(g0, TROWS)], scale_v)
    pltpu.sync_copy(bias_hbm.at[pl.ds(g0, TROWS)], bias_v)
    nch = lax.reduce_max(n16[...], (0,))
    plsc.subcore_barrier()

    rows = (rows0, rows1, rows2)
    sems = (sem0, sem1, sem2)

    @pl.when(nch > 0)
    def _():
        pltpu.async_copy(t_hbm.at[esrc_v.at[0]], rows0, sem0)

    @pl.when(nch > 1)
    def _():
        pltpu.async_copy(t_hbm.at[esrc_v.at[1]], rows1, sem1)

    def triple(q, _):
        for j in range(3):
            cc = q * 3 + j
            jn = (j + 2) % 3

            @pl.when(cc < nch)
            def _(j=j, jn=jn, cc=cc):
                @pl.when(cc + 2 < nch)
                def _():
                    pltpu.async_copy(
                        t_hbm.at[esrc_v.at[cc + 2]], rows[jn], sems[jn])

                pltpu.make_async_copy(
                    t_hbm.at[esrc_v.at[cc]], rows[j], sems[j]).wait()
                pltpu.sync_copy(rows[j], accsp.at[edst_v.at[cc]], add=True)
        return 0

    lax.fori_loop(0, (nch + 2) // 3, triple, 0)
    plsc.subcore_barrier()

    pltpu.sync_copy(accsp.at[pl.ds(lo, TROWS)], accbuf)

    def srow(rb, _):
        s16 = scale_v[pl.ds(rb * 16, 16)]
        for j in range(16):
            r = rb * 16 + j
            sc = s16[j]
            for g in range(4):
                accbuf[r, pl.ds(g * 16, 16)] = (
                    accbuf[r, pl.ds(g * 16, 16)] * sc
                    + bias_v[r, pl.ds(g * 16, 16)]
                )
        return 0
    lax.fori_loop(0, TROWS // 16, srow, 0)
    pltpu.sync_copy(accbuf, tout_hbm.at[pl.ds(g0, TROWS)])


_sc_conv = pl.kernel(
    _sc_conv_body,
    out_type=jax.ShapeDtypeStruct((NPAD, C), jnp.float32),
    mesh=_MESH,
    compiler_params=pltpu.CompilerParams(needs_layout_passes=False, use_tc_tiling_on_sc=False),
    scratch_types=[
        pltpu.VMEM((CHMAX, CHUNK), jnp.int32),
        pltpu.VMEM((CHMAX, CHUNK), jnp.int32),
        pltpu.VMEM((CHUNK, C), jnp.float32),
        pltpu.VMEM((CHUNK, C), jnp.float32),
        pltpu.VMEM((CHUNK, C), jnp.float32),
        pltpu.VMEM((TROWS, C), jnp.float32),
        pltpu.VMEM((TROWS, C), jnp.float32),
        pltpu.VMEM((TROWS,), jnp.float32),
        pltpu.VMEM((16,), jnp.int32),
        pltpu.VMEM_SHARED((NACC, C), jnp.float32),
        pltpu.SemaphoreType.DMA,
        pltpu.SemaphoreType.DMA,
        pltpu.SemaphoreType.DMA,
    ],
)


def kernel(x, edge_index, train_idx, labels, W1, b1, W2, b2):
    src = edge_index[0].astype(jnp.int32)
    dst = edge_index[1].astype(jnp.int32)
    tr = jnp.concatenate(
        [train_idx.astype(jnp.int32),
         jnp.full((NTR - train_idx.shape[0],), TRPAD, jnp.int32)])
    lab = jnp.concatenate(
        [labels.astype(jnp.int32),
         jnp.zeros((NTR - labels.shape[0],), jnp.int32)])

    p = _mlp_softmax(x, W1, b1, W2, b2)
    p_pad = jnp.pad(p, ((0, NPAD - N), (0, 0)))

    esrc, edst, nch, scale1, scale2, bias1, t0 = _sc_setup(
        src, dst, tr, lab, p_pad)
    esrc3 = esrc.reshape(NSC * NT, CHMAX, CHUNK)
    edst3 = edst.reshape(NSC * NT, CHMAX, CHUNK)

    t = t0
    for _ in range(DEPTH):
        t = _sc_conv(t, esrc3, edst3, nch, scale1, bias1)

    h0b, bias2 = _transition(p_pad, t)
    t = h0b
    for _ in range(DEPTH):
        t = _sc_conv(t, esrc3, edst3, nch, scale2, bias2)

    return _finalize(t[:N])
